# baseline (speedup 1.0000x reference)
import functools

import jax
import jax.numpy as jnp
from jax import lax
from jax.experimental import pallas as pl
from jax.experimental.pallas import tpu as pltpu
from jax.experimental.pallas import tpu_sc as plsc

_FEAT_DIM = 128
_BATCH = 4096
_LAMBDA_C = 1.0
_LANES = 16

_info = plsc.get_sparse_core_info()
_NC, _NS = _info.num_cores, _info.num_subcores
_NW = _NC * _NS
_BPW = _BATCH // _NW
_CHUNKS = (96, 32)


def _center_loss_partials(feat, label, centers):
  mesh = plsc.VectorSubcoreMesh(core_axis_name="c", subcore_axis_name="s")

  @functools.partial(
      pl.kernel,
      mesh=mesh,
      out_type=jax.ShapeDtypeStruct((_NW, _LANES), jnp.float32),
      scratch_types=[
          pltpu.VMEM((_BPW,), jnp.int32),
          pltpu.VMEM((_BPW, _FEAT_DIM), jnp.float32),
          pltpu.VMEM((_BPW, _FEAT_DIM), jnp.float32),
          pltpu.VMEM((_LANES,), jnp.float32),
          pltpu.SemaphoreType.DMA,
          pltpu.SemaphoreType.DMA,
          pltpu.SemaphoreType.DMA,
      ],
  )
  def k(feat_hbm, label_hbm, centers_hbm, out_hbm,
        idx_v, feat_v, rows_v, acc_v, fsem, gs0, gs1):
    wid = lax.axis_index("s") * _NC + lax.axis_index("c")
    fcopy = pltpu.async_copy(feat_hbm.at[wid], feat_v, fsem)
    pltpu.sync_copy(label_hbm.at[wid], idx_v)
    gsems = (gs0, gs1)
    bases = (0, _CHUNKS[0])
    gathers = [
        pltpu.async_copy(
            centers_hbm.at[idx_v.at[pl.ds(bases[c], _CHUNKS[c])]],
            rows_v.at[pl.ds(bases[c], _CHUNKS[c])], gsems[c])
        for c in range(2)
    ]
    fcopy.wait()

    acc = (jnp.zeros((_LANES,), jnp.float32), jnp.zeros((_LANES,), jnp.float32))
    for c in range(2):
      gathers[c].wait()
      base = bases[c]

      def body(r, a, base=base):
        a = list(a)
        for d in range(_FEAT_DIM // _LANES):
          x = feat_v[base + r, pl.ds(d * _LANES, _LANES)]
          y = rows_v[base + r, pl.ds(d * _LANES, _LANES)]
          diff = x - y
          a[d % 2] = a[d % 2] + diff * diff
        return tuple(a)

      acc = lax.fori_loop(0, _CHUNKS[c], body, acc)

    acc_v[...] = acc[0] + acc[1]
    pltpu.sync_copy(acc_v, out_hbm.at[wid])

  return k(feat, label, centers)


def kernel(feat, label, centers):
  label = label.astype(jnp.int32).reshape(_NW, _BPW)
  feat_r = feat.reshape(_NW, _BPW, _FEAT_DIM)
  partials = _center_loss_partials(feat_r, label, centers)
  return _LAMBDA_C / 2.0 / _BATCH * jnp.sqrt(jnp.sum(partials))


# speedup vs baseline: 1.0058x; 1.0058x over previous
import functools

import jax
import jax.numpy as jnp
from jax import lax
from jax.experimental import pallas as pl
from jax.experimental.pallas import tpu as pltpu
from jax.experimental.pallas import tpu_sc as plsc

_FEAT_DIM = 128
_BATCH = 4096
_LAMBDA_C = 1.0
_LANES = 16

_info = plsc.get_sparse_core_info()
_NC, _NS = _info.num_cores, _info.num_subcores
_NW = _NC * _NS
_BPW = _BATCH // _NW
_CHUNKS = (112, 16)


def _center_loss_partials(feat, label, centers):
  mesh = plsc.VectorSubcoreMesh(core_axis_name="c", subcore_axis_name="s")

  @functools.partial(
      pl.kernel,
      mesh=mesh,
      out_type=jax.ShapeDtypeStruct((_NW, _LANES), jnp.float32),
      scratch_types=[
          pltpu.VMEM((_BPW,), jnp.int32),
          pltpu.VMEM((_BPW, _FEAT_DIM), jnp.float32),
          pltpu.VMEM((_BPW, _FEAT_DIM), jnp.float32),
          pltpu.VMEM((_LANES,), jnp.float32),
          pltpu.SemaphoreType.DMA,
          pltpu.SemaphoreType.DMA,
          pltpu.SemaphoreType.DMA,
      ],
  )
  def k(feat_hbm, label_hbm, centers_hbm, out_hbm,
        idx_v, feat_v, rows_v, acc_v, fsem, gs0, gs1):
    wid = lax.axis_index("s") * _NC + lax.axis_index("c")
    fcopy = pltpu.async_copy(feat_hbm.at[wid], feat_v, fsem)
    pltpu.sync_copy(label_hbm.at[wid], idx_v)
    gsems = (gs0, gs1)
    bases = (0, _CHUNKS[0])
    gathers = [
        pltpu.async_copy(
            centers_hbm.at[idx_v.at[pl.ds(bases[c], _CHUNKS[c])]],
            rows_v.at[pl.ds(bases[c], _CHUNKS[c])], gsems[c])
        for c in range(2)
    ]
    fcopy.wait()

    acc = jnp.zeros((_LANES,), jnp.float32)
    for c in range(2):
      gathers[c].wait()
      base = bases[c]

      def body(r, a, base=base):
        for d in range(_FEAT_DIM // _LANES):
          x = feat_v[base + r, pl.ds(d * _LANES, _LANES)]
          y = rows_v[base + r, pl.ds(d * _LANES, _LANES)]
          diff = x - y
          a = a + diff * diff
        return a

      acc = lax.fori_loop(0, _CHUNKS[c], body, acc)

    acc_v[...] = acc
    pltpu.sync_copy(acc_v, out_hbm.at[wid])

  return k(feat, label, centers)


def kernel(feat, label, centers):
  label = label.astype(jnp.int32).reshape(_NW, _BPW)
  feat_r = feat.reshape(_NW, _BPW, _FEAT_DIM)
  partials = _center_loss_partials(feat_r, label, centers)
  return _LAMBDA_C / 2.0 / _BATCH * jnp.sqrt(jnp.sum(partials))


# speedup vs baseline: 1.0071x; 1.0013x over previous
import functools

import jax
import jax.numpy as jnp
from jax import lax
from jax.experimental import pallas as pl
from jax.experimental.pallas import tpu as pltpu
from jax.experimental.pallas import tpu_sc as plsc

_FEAT_DIM = 128
_BATCH = 4096
_LAMBDA_C = 1.0
_LANES = 16

_info = plsc.get_sparse_core_info()
_NC, _NS = _info.num_cores, _info.num_subcores
_NW = _NC * _NS
_BPW = _BATCH // _NW
_CHUNKS = (120, 8)


def _center_loss_partials(feat, label, centers):
  mesh = plsc.VectorSubcoreMesh(core_axis_name="c", subcore_axis_name="s")

  @functools.partial(
      pl.kernel,
      mesh=mesh,
      out_type=jax.ShapeDtypeStruct((_NW, _LANES), jnp.float32),
      scratch_types=[
          pltpu.VMEM((_BPW,), jnp.int32),
          pltpu.VMEM((_BPW, _FEAT_DIM), jnp.float32),
          pltpu.VMEM((_BPW, _FEAT_DIM), jnp.float32),
          pltpu.VMEM((_LANES,), jnp.float32),
          pltpu.SemaphoreType.DMA,
          pltpu.SemaphoreType.DMA,
          pltpu.SemaphoreType.DMA,
      ],
  )
  def k(feat_hbm, label_hbm, centers_hbm, out_hbm,
        idx_v, feat_v, rows_v, acc_v, fsem, gs0, gs1):
    wid = lax.axis_index("s") * _NC + lax.axis_index("c")
    fcopy = pltpu.async_copy(feat_hbm.at[wid], feat_v, fsem)
    pltpu.sync_copy(label_hbm.at[wid], idx_v)
    gsems = (gs0, gs1)
    bases = (0, _CHUNKS[0])
    gathers = [
        pltpu.async_copy(
            centers_hbm.at[idx_v.at[pl.ds(bases[c], _CHUNKS[c])]],
            rows_v.at[pl.ds(bases[c], _CHUNKS[c])], gsems[c])
        for c in range(2)
    ]
    fcopy.wait()

    acc = jnp.zeros((_LANES,), jnp.float32)
    for c in range(2):
      gathers[c].wait()
      base = bases[c]

      def body(r, a, base=base):
        for d in range(_FEAT_DIM // _LANES):
          x = feat_v[base + r, pl.ds(d * _LANES, _LANES)]
          y = rows_v[base + r, pl.ds(d * _LANES, _LANES)]
          diff = x - y
          a = a + diff * diff
        return a

      acc = lax.fori_loop(0, _CHUNKS[c], body, acc)

    acc_v[...] = acc
    pltpu.sync_copy(acc_v, out_hbm.at[wid])

  return k(feat, label, centers)


def kernel(feat, label, centers):
  label = label.astype(jnp.int32).reshape(_NW, _BPW)
  feat_r = feat.reshape(_NW, _BPW, _FEAT_DIM)
  partials = _center_loss_partials(feat_r, label, centers)
  return _LAMBDA_C / 2.0 / _BATCH * jnp.sqrt(jnp.sum(partials))
